# baseline (speedup 1.0000x reference)
import functools

import jax
import jax.numpy as jnp
from jax import lax
from jax.experimental import pallas as pl
from jax.experimental.pallas import tpu as pltpu
from jax.experimental.pallas import tpu_sc as plsc

NC = 2
NS = 16
NW = NC * NS

D = 1024
SEQ = 4096
CROWS = 32
NCHUNK = 16
VPR = D // 16


def _body(ids_hbm, wte_hbm, wpe_hbm, out_hbm,
          idx_v, wbuf, pbuf, gsem0, gsem1, ssem0, ssem1):
    wid = lax.axis_index("s") * NC + lax.axis_index("c")
    base = wid * (NCHUNK * CROWS)
    pos_base = lax.rem(base, SEQ)

    pltpu.sync_copy(ids_hbm.at[wid], idx_v)

    gsems = [gsem0, gsem1]
    ssems = [ssem0, ssem1]
    gathers = [None, None]
    stores = [None, None]

    gathers[0] = pltpu.make_async_copy(
        wte_hbm.at[idx_v.at[0]], wbuf.at[0], gsems[0])
    gathers[0].start()

    for c in range(NCHUNK):
        s = c % 2
        n = (c + 1) % 2
        if c + 1 < NCHUNK:
            if stores[n] is not None:
                stores[n].wait()
                stores[n] = None
            gathers[n] = pltpu.make_async_copy(
                wte_hbm.at[idx_v.at[c + 1]], wbuf.at[n], gsems[n])
            gathers[n].start()

        pltpu.sync_copy(wpe_hbm.at[pl.ds(pos_base + c * CROWS, CROWS)], pbuf)
        gathers[s].wait()

        def add_body(j, carry, s=s):
            r = j // VPR
            k = (j % VPR) * 16
            wbuf[s, r, pl.ds(k, 16)] = (
                wbuf[s, r, pl.ds(k, 16)] + pbuf[r, pl.ds(k, 16)])
            return carry

        lax.fori_loop(0, CROWS * VPR, add_body, 0)

        if stores[s] is not None:
            stores[s].wait()
        stores[s] = pltpu.make_async_copy(
            wbuf.at[s], out_hbm.at[pl.ds(base + c * CROWS, CROWS)], ssems[s])
        stores[s].start()

    for st in stores:
        if st is not None:
            st.wait()


@jax.jit
def _sc_embed(ids3d, wte, wpe):
    mesh = plsc.VectorSubcoreMesh(
        core_axis_name="c", subcore_axis_name="s",
        num_cores=NC, num_subcores=NS)
    f = pl.kernel(
        _body,
        out_type=jax.ShapeDtypeStruct((NW * NCHUNK * CROWS, D), jnp.float32),
        mesh=mesh,
        scratch_types=[
            pltpu.VMEM((NCHUNK, CROWS), jnp.int32),
            pltpu.VMEM((2, CROWS, D), jnp.float32),
            pltpu.VMEM((CROWS, D), jnp.float32),
            pltpu.SemaphoreType.DMA,
            pltpu.SemaphoreType.DMA,
            pltpu.SemaphoreType.DMA,
            pltpu.SemaphoreType.DMA,
        ],
    )
    return f(ids3d, wte, wpe)


def kernel(input_ids, wte, wpe):
    B, L = input_ids.shape
    ids3d = input_ids.reshape(NW, NCHUNK, CROWS)
    out = _sc_embed(ids3d, wte, wpe)
    return out.reshape(B, L, D)
